# Initial kernel scaffold; baseline (speedup 1.0000x reference)
import functools

import jax
import jax.numpy as jnp
from jax import lax
from jax.experimental import pallas as pl
from jax.experimental.pallas import tpu as pltpu
from jax.experimental.pallas import tpu_sc as plsc

N = 10000
E = 320000
D = 128
DC = 64

NC = 2
NS = 16
NW = NC * NS
EW = E // NW
C = 80
NCH = EW // C
RPT = N // NS
ZR = 125

_sc_mesh = plsc.VectorSubcoreMesh(
    core_axis_name="c", subcore_axis_name="s", num_cores=NC, num_subcores=NS)


def _spmm_sc(table, src_r, dst_r, w_r):
    @functools.partial(
        pl.kernel,
        out_type=jax.ShapeDtypeStruct((NC, N, D), jnp.float32),
        mesh=_sc_mesh,
        scratch_types=[
            pltpu.VMEM_SHARED((N, D), jnp.float32),
            pltpu.VMEM((NCH, C), jnp.int32),
            pltpu.VMEM((NCH, C), jnp.int32),
            pltpu.VMEM((NCH, C), jnp.float32),
            pltpu.VMEM((C, D), jnp.float32),
            pltpu.VMEM((ZR, D), jnp.float32),
        ],
    )
    def spmm_kernel(table_hbm, src_hbm, dst_hbm, w_hbm, out_hbm,
                    acc_sh, src_v, dst_v, w_v, rows_v, zb_v):
        c = lax.axis_index("c")
        s = lax.axis_index("s")
        wid = c * NS + s

        pltpu.sync_copy(src_hbm.at[wid], src_v)
        pltpu.sync_copy(dst_hbm.at[wid], dst_v)
        pltpu.sync_copy(w_hbm.at[wid], w_v)

        @pl.loop(0, ZR)
        def _(r):
            for dd in range(D // 16):
                zb_v[r, pl.ds(dd * 16, 16)] = jnp.zeros((16,), jnp.float32)

        for k in range(RPT // ZR):
            pltpu.sync_copy(zb_v, acc_sh.at[pl.ds(s * RPT + k * ZR, ZR)])
        plsc.subcore_barrier()

        @pl.loop(0, NCH)
        def _(j):
            pltpu.sync_copy(table_hbm.at[src_v.at[j]], rows_v)

            @pl.loop(0, C)
            def _(e):
                w_s = w_v[j, e]
                spl = jnp.full((16,), w_s, jnp.float32)
                for dd in range(D // 16):
                    sl = pl.ds(dd * 16, 16)
                    rows_v[e, sl] = rows_v[e, sl] * spl

            pltpu.sync_copy(rows_v, acc_sh.at[dst_v.at[j]], add=True)

        plsc.subcore_barrier()

        pltpu.sync_copy(acc_sh.at[pl.ds(s * RPT, RPT)],
                        out_hbm.at[c, pl.ds(s * RPT, RPT)])

    return spmm_kernel(table, src_r, dst_r, w_r)


_BM = 500


def _mm1_body(x_ref, w_ref, o_ref):
    o_ref[...] = jnp.dot(x_ref[...], w_ref[...],
                         preferred_element_type=jnp.float32)


def _mm1(x, W1):
    return pl.pallas_call(
        _mm1_body,
        grid=(N // _BM,),
        in_specs=[
            pl.BlockSpec((_BM, D), lambda i: (i, 0)),
            pl.BlockSpec((D, D), lambda i: (0, 0)),
        ],
        out_specs=pl.BlockSpec((_BM, D), lambda i: (i, 0)),
        out_shape=jax.ShapeDtypeStruct((N, D), jnp.float32),
    )(x, W1)


def _combine_relu_body(p_ref, b_ref, o_ref):
    o_ref[...] = jnp.maximum(p_ref[0] + p_ref[1] + b_ref[...], 0.0)


def _combine_relu(p, b1):
    return pl.pallas_call(
        _combine_relu_body,
        grid=(N // _BM,),
        in_specs=[
            pl.BlockSpec((NC, _BM, D), lambda i: (0, i, 0)),
            pl.BlockSpec((1, D), lambda i: (0, 0)),
        ],
        out_specs=pl.BlockSpec((_BM, D), lambda i: (i, 0)),
        out_shape=jax.ShapeDtypeStruct((N, D), jnp.float32),
    )(p, b1.reshape(1, D))


def _final_body(q_ref, w_ref, b_ref, o_ref):
    t = q_ref[0] + q_ref[1]
    o = jnp.dot(t, w_ref[...], preferred_element_type=jnp.float32) + b_ref[...]
    m = jnp.max(o, axis=1, keepdims=True)
    ex = jnp.exp(o - m)
    lse = jnp.log(jnp.sum(ex, axis=1, keepdims=True)) + m
    o_ref[...] = o - lse


def _final(q, W2, b2):
    return pl.pallas_call(
        _final_body,
        grid=(N // _BM,),
        in_specs=[
            pl.BlockSpec((NC, _BM, D), lambda i: (0, i, 0)),
            pl.BlockSpec((D, DC), lambda i: (0, 0)),
            pl.BlockSpec((1, DC), lambda i: (0, 0)),
        ],
        out_specs=pl.BlockSpec((_BM, DC), lambda i: (i, 0)),
        out_shape=jax.ShapeDtypeStruct((N, DC), jnp.float32),
    )(q, W2, b2.reshape(1, DC))


def kernel(x, edge_index, edge_weight, W1, b1, W2, b2):
    src_r = edge_index[0].reshape(NW, NCH, C)
    dst_r = edge_index[1].reshape(NW, NCH, C)
    w_r = edge_weight.reshape(NW, NCH, C)

    support = _mm1(x, W1)
    p = _spmm_sc(support, src_r, dst_r, w_r)
    h = _combine_relu(p, b1)
    q = _spmm_sc(h, src_r, dst_r, w_r)
    return _final(q, W2, b2)


# speedup vs baseline: 7.0797x; 7.0797x over previous
import functools

import jax
import jax.numpy as jnp
from jax import lax
from jax.experimental import pallas as pl
from jax.experimental.pallas import tpu as pltpu
from jax.experimental.pallas import tpu_sc as plsc

N = 10000
E = 320000
D = 128
DC = 64

NC = 2
NS = 16
NW = NC * NS
C = 128
NCH = 80
EP = NW * NCH * C
RPT = N // NS

_sc_mesh = plsc.VectorSubcoreMesh(
    core_axis_name="c", subcore_axis_name="s", num_cores=NC, num_subcores=NS)


def _spmm_sc(table, src_r, dst_r, w_r):
    @functools.partial(
        pl.kernel,
        out_type=jax.ShapeDtypeStruct((NC, N, D), jnp.float32),
        mesh=_sc_mesh,
        scratch_types=[
            pltpu.VMEM_SHARED((N, D), jnp.float32),
            pltpu.VMEM((NCH, C), jnp.int32),
            pltpu.VMEM((NCH, C), jnp.int32),
            pltpu.VMEM((NCH, C), jnp.float32),
            pltpu.VMEM((C, D), jnp.float32),
        ],
    )
    def spmm_kernel(table_hbm, src_hbm, dst_hbm, w_hbm, out_hbm,
                    acc_sh, src_v, dst_v, w_v, rows_v):
        c = lax.axis_index("c")
        s = lax.axis_index("s")
        wid = c * NS + s

        pltpu.sync_copy(src_hbm.at[wid], src_v)
        pltpu.sync_copy(dst_hbm.at[wid], dst_v)
        pltpu.sync_copy(w_hbm.at[wid], w_v)

        @pl.loop(0, C)
        def _(r):
            for dd in range(D // 16):
                rows_v[r, pl.ds(dd * 16, 16)] = jnp.zeros((16,), jnp.float32)

        for k in range(RPT // C):
            pltpu.sync_copy(rows_v, acc_sh.at[pl.ds(s * RPT + k * C, C)])
        rem = RPT % C
        if rem:
            pltpu.sync_copy(rows_v.at[pl.ds(0, rem)],
                            acc_sh.at[pl.ds(s * RPT + (RPT // C) * C, rem)])
        plsc.subcore_barrier()

        @pl.loop(0, NCH)
        def _(j):
            pltpu.sync_copy(table_hbm.at[src_v.at[j]], rows_v)

            @pl.loop(0, C // 16)
            def _(g):
                wv = w_v[j, pl.ds(g * 16, 16)]
                for k in range(16):
                    spl = jnp.full((16,), wv[k], jnp.float32)
                    e = g * 16 + k
                    for dd in range(D // 16):
                        sl = pl.ds(dd * 16, 16)
                        rows_v[e, sl] = rows_v[e, sl] * spl

            pltpu.sync_copy(rows_v, acc_sh.at[dst_v.at[j]], add=True)

        plsc.subcore_barrier()

        WO = 624
        pltpu.sync_copy(acc_sh.at[pl.ds(s * WO, WO)],
                        out_hbm.at[c, pl.ds(s * WO, WO)])

        @pl.when(s == NS - 1)
        def _():
            pltpu.sync_copy(acc_sh.at[pl.ds(NS * WO, N - NS * WO)],
                            out_hbm.at[c, pl.ds(NS * WO, N - NS * WO)])

    return spmm_kernel(table, src_r, dst_r, w_r)


_BM = 1000


def _mm1_body(x_ref, w_ref, o_ref):
    o_ref[...] = jnp.dot(x_ref[...], w_ref[...],
                         preferred_element_type=jnp.float32)


def _mm1(x, W1):
    return pl.pallas_call(
        _mm1_body,
        grid=(N // _BM,),
        in_specs=[
            pl.BlockSpec((_BM, D), lambda i: (i, 0)),
            pl.BlockSpec((D, D), lambda i: (0, 0)),
        ],
        out_specs=pl.BlockSpec((_BM, D), lambda i: (i, 0)),
        out_shape=jax.ShapeDtypeStruct((N, D), jnp.float32),
    )(x, W1)


def _combine_relu_body(p_ref, b_ref, o_ref):
    o_ref[...] = jnp.maximum(p_ref[0] + p_ref[1] + b_ref[...], 0.0)


def _combine_relu(p, b1):
    return pl.pallas_call(
        _combine_relu_body,
        grid=(N // _BM,),
        in_specs=[
            pl.BlockSpec((NC, _BM, D), lambda i: (0, i, 0)),
            pl.BlockSpec((1, D), lambda i: (0, 0)),
        ],
        out_specs=pl.BlockSpec((_BM, D), lambda i: (i, 0)),
        out_shape=jax.ShapeDtypeStruct((N, D), jnp.float32),
    )(p, b1.reshape(1, D))


def _final_body(q_ref, w_ref, b_ref, o_ref):
    t = q_ref[0] + q_ref[1]
    o = jnp.dot(t, w_ref[...], preferred_element_type=jnp.float32) + b_ref[...]
    m = jnp.max(o, axis=1, keepdims=True)
    ex = jnp.exp(o - m)
    lse = jnp.log(jnp.sum(ex, axis=1, keepdims=True)) + m
    o_ref[...] = o - lse


def _final(q, W2, b2):
    return pl.pallas_call(
        _final_body,
        grid=(N // _BM,),
        in_specs=[
            pl.BlockSpec((NC, _BM, D), lambda i: (0, i, 0)),
            pl.BlockSpec((D, DC), lambda i: (0, 0)),
            pl.BlockSpec((1, DC), lambda i: (0, 0)),
        ],
        out_specs=pl.BlockSpec((_BM, DC), lambda i: (i, 0)),
        out_shape=jax.ShapeDtypeStruct((N, DC), jnp.float32),
    )(q, W2, b2.reshape(1, DC))


def kernel(x, edge_index, edge_weight, W1, b1, W2, b2):
    pad = EP - E
    pad_idx = (jnp.arange(pad, dtype=jnp.int32) * 8) % N
    src_r = jnp.concatenate([edge_index[0], pad_idx]).reshape(NW, NCH, C)
    dst_r = jnp.concatenate([edge_index[1], pad_idx]).reshape(NW, NCH, C)
    w_r = jnp.concatenate(
        [edge_weight, jnp.zeros((pad,), jnp.float32)]).reshape(NW, NCH, C)

    support = _mm1(x, W1)
    p = _spmm_sc(support, src_r, dst_r, w_r)
    h = _combine_relu(p, b1)
    q = _spmm_sc(h, src_r, dst_r, w_r)
    return _final(q, W2, b2)


# speedup vs baseline: 9.9368x; 1.4036x over previous
import functools

import jax
import jax.numpy as jnp
from jax import lax
from jax.experimental import pallas as pl
from jax.experimental.pallas import tpu as pltpu
from jax.experimental.pallas import tpu_sc as plsc

N = 10000
E = 320000
D = 128
DC = 64

NC = 2
NS = 16
NW = NC * NS
C = 128
NCH = 80
EP = NW * NCH * C
RPT = N // NS

_sc_mesh = plsc.VectorSubcoreMesh(
    core_axis_name="c", subcore_axis_name="s", num_cores=NC, num_subcores=NS)


def _spmm_sc(table, e3):
    @functools.partial(
        pl.kernel,
        out_type=jax.ShapeDtypeStruct((NC, N, D), jnp.float32),
        mesh=_sc_mesh,
        scratch_types=[
            pltpu.VMEM_SHARED((N, D), jnp.float32),
            pltpu.VMEM((3, C), jnp.int32),
            pltpu.VMEM((3, C), jnp.int32),
            pltpu.VMEM((C,), jnp.int32),
            pltpu.VMEM((C,), jnp.int32),
            pltpu.VMEM((C, D), jnp.float32),
            pltpu.VMEM((C, D), jnp.float32),
            pltpu.SemaphoreType.DMA,
            pltpu.SemaphoreType.DMA,
            pltpu.SemaphoreType.DMA,
            pltpu.SemaphoreType.DMA,
            pltpu.SemaphoreType.DMA,
            pltpu.SemaphoreType.DMA,
        ],
    )
    def spmm_kernel(table_hbm, e3_hbm, out_hbm,
                    acc_sh, e3_a, e3_b, dc_a, dc_b, rows_a, rows_b,
                    si_a, si_b, sg_a, sg_b, ss_a, ss_b):
        c = lax.axis_index("c")
        s = lax.axis_index("s")
        wid = c * NS + s

        @pl.loop(0, C)
        def _(r):
            for dd in range(D // 16):
                rows_a[r, pl.ds(dd * 16, 16)] = jnp.zeros((16,), jnp.float32)

        for k in range(RPT // C):
            pltpu.sync_copy(rows_a, acc_sh.at[pl.ds(s * RPT + k * C, C)])
        rem = RPT % C
        if rem:
            pltpu.sync_copy(rows_a.at[pl.ds(0, rem)],
                            acc_sh.at[pl.ds(s * RPT + (RPT // C) * C, rem)])
        plsc.subcore_barrier()

        def process(j, e3_v, dc_v, rows_v, sg, ss, si):
            pltpu.make_async_copy(table_hbm.at[e3_v.at[0]], rows_v, sg).wait()
            for g in range(C // 16):
                sl = pl.ds(g * 16, 16)
                dc_v[sl] = e3_v[1, sl]

            @pl.loop(0, C // 16)
            def _(g):
                wv = lax.bitcast_convert_type(
                    e3_v[2, pl.ds(g * 16, 16)], jnp.float32)
                for k in range(16):
                    spl = jnp.full((16,), wv[k], jnp.float32)
                    e = g * 16 + k
                    for dd in range(D // 16):
                        sl2 = pl.ds(dd * 16, 16)
                        rows_v[e, sl2] = rows_v[e, sl2] * spl

            pltpu.async_copy(rows_v, acc_sh.at[dc_v], ss, add=True)

            @pl.when(j + 2 < NCH)
            def _():
                pltpu.async_copy(e3_hbm.at[wid, j + 2], e3_v, si)

        def refill_gather(j, e3_v, dc_v, rows_v, sg, ss, si):
            @pl.when(j < NCH)
            def _():
                pltpu.make_async_copy(rows_v, acc_sh.at[dc_v], ss).wait()
                pltpu.make_async_copy(e3_hbm.at[wid, 0], e3_v, si).wait()
                pltpu.async_copy(table_hbm.at[e3_v.at[0]], rows_v, sg)

        pltpu.async_copy(e3_hbm.at[wid, 0], e3_a, si_a)
        pltpu.async_copy(e3_hbm.at[wid, 1], e3_b, si_b)
        pltpu.make_async_copy(e3_hbm.at[wid, 0], e3_a, si_a).wait()
        pltpu.async_copy(table_hbm.at[e3_a.at[0]], rows_a, sg_a)
        pltpu.make_async_copy(e3_hbm.at[wid, 1], e3_b, si_b).wait()
        pltpu.async_copy(table_hbm.at[e3_b.at[0]], rows_b, sg_b)

        @pl.loop(0, NCH // 2)
        def _(it):
            j0 = it * 2
            j1 = j0 + 1
            process(j0, e3_a, dc_a, rows_a, sg_a, ss_a, si_a)
            process(j1, e3_b, dc_b, rows_b, sg_b, ss_b, si_b)
            refill_gather(j0 + 2, e3_a, dc_a, rows_a, sg_a, ss_a, si_a)
            refill_gather(j1 + 2, e3_b, dc_b, rows_b, sg_b, ss_b, si_b)

        pltpu.make_async_copy(rows_a, acc_sh.at[dc_a], ss_a).wait()
        pltpu.make_async_copy(rows_b, acc_sh.at[dc_b], ss_b).wait()
        plsc.subcore_barrier()

        WO = 624
        pltpu.sync_copy(acc_sh.at[pl.ds(s * WO, WO)],
                        out_hbm.at[c, pl.ds(s * WO, WO)])

        @pl.when(s == NS - 1)
        def _():
            pltpu.sync_copy(acc_sh.at[pl.ds(NS * WO, N - NS * WO)],
                            out_hbm.at[c, pl.ds(NS * WO, N - NS * WO)])

    return spmm_kernel(table, e3)


_BM = 1000


def _mm1_body(x_ref, w_ref, o_ref):
    o_ref[...] = jnp.dot(x_ref[...], w_ref[...],
                         preferred_element_type=jnp.float32)


def _mm1(x, W1):
    return pl.pallas_call(
        _mm1_body,
        grid=(N // _BM,),
        in_specs=[
            pl.BlockSpec((_BM, D), lambda i: (i, 0)),
            pl.BlockSpec((D, D), lambda i: (0, 0)),
        ],
        out_specs=pl.BlockSpec((_BM, D), lambda i: (i, 0)),
        out_shape=jax.ShapeDtypeStruct((N, D), jnp.float32),
    )(x, W1)


def _combine_relu_body(p_ref, b_ref, o_ref):
    o_ref[...] = jnp.maximum(p_ref[0] + p_ref[1] + b_ref[...], 0.0)


def _combine_relu(p, b1):
    return pl.pallas_call(
        _combine_relu_body,
        grid=(N // _BM,),
        in_specs=[
            pl.BlockSpec((NC, _BM, D), lambda i: (0, i, 0)),
            pl.BlockSpec((1, D), lambda i: (0, 0)),
        ],
        out_specs=pl.BlockSpec((_BM, D), lambda i: (i, 0)),
        out_shape=jax.ShapeDtypeStruct((N, D), jnp.float32),
    )(p, b1.reshape(1, D))


def _final_body(q_ref, w_ref, b_ref, o_ref):
    t = q_ref[0] + q_ref[1]
    o = jnp.dot(t, w_ref[...], preferred_element_type=jnp.float32) + b_ref[...]
    m = jnp.max(o, axis=1, keepdims=True)
    ex = jnp.exp(o - m)
    lse = jnp.log(jnp.sum(ex, axis=1, keepdims=True)) + m
    o_ref[...] = o - lse


def _final(q, W2, b2):
    return pl.pallas_call(
        _final_body,
        grid=(N // _BM,),
        in_specs=[
            pl.BlockSpec((NC, _BM, D), lambda i: (0, i, 0)),
            pl.BlockSpec((D, DC), lambda i: (0, 0)),
            pl.BlockSpec((1, DC), lambda i: (0, 0)),
        ],
        out_specs=pl.BlockSpec((_BM, DC), lambda i: (i, 0)),
        out_shape=jax.ShapeDtypeStruct((N, DC), jnp.float32),
    )(q, W2, b2.reshape(1, DC))


def kernel(x, edge_index, edge_weight, W1, b1, W2, b2):
    pad = EP - E
    pad_idx = (jnp.arange(pad, dtype=jnp.int32) * 8) % N
    src_p = jnp.concatenate([edge_index[0], pad_idx]).reshape(NW, NCH, 1, C)
    dst_p = jnp.concatenate([edge_index[1], pad_idx]).reshape(NW, NCH, 1, C)
    w_bits = lax.bitcast_convert_type(
        jnp.concatenate([edge_weight, jnp.zeros((pad,), jnp.float32)]),
        jnp.int32).reshape(NW, NCH, 1, C)
    e3 = jnp.concatenate([src_p, dst_p, w_bits], axis=2)

    support = _mm1(x, W1)
    p = _spmm_sc(support, e3)
    h = _combine_relu(p, b1)
    q = _spmm_sc(h, e3)
    return _final(q, W2, b2)
